# baseline (device time: 24436 ns/iter reference)
import jax
import jax.numpy as jnp
from jax import lax
from jax.experimental import pallas as pl
from jax.experimental.pallas import tpu as pltpu

N_DEV = 8
B, S_LOC, D_MODEL = 2, 128, 512
HQ, DH = 4, 64
WINDOW = 128
SCALE = 0.125


def kernel(x, Wq, K_ext, V_ext, Wo):
    def body(x_ref, wq_ref, k_ref, v_ref, wo_ref, out_ref,
             kl_ref, kr_ref, vl_ref, vr_ref, send_sems, recv_sems):
        my = lax.axis_index("i")
        left = lax.rem(my + N_DEV - 1, N_DEV)
        right = lax.rem(my + 1, N_DEV)

        barrier_sem = pltpu.get_barrier_semaphore()
        for nbr in (left, right):
            pl.semaphore_signal(
                barrier_sem, inc=1,
                device_id=(nbr,), device_id_type=pl.DeviceIdType.MESH,
            )
        pl.semaphore_wait(barrier_sem, 2)

        rdmas = []
        for idx, (src, dst, tgt) in enumerate((
            (k_ref, kl_ref, right),
            (k_ref, kr_ref, left),
            (v_ref, vl_ref, right),
            (v_ref, vr_ref, left),
        )):
            r = pltpu.make_async_remote_copy(
                src_ref=src, dst_ref=dst,
                send_sem=send_sems.at[idx], recv_sem=recv_sems.at[idx],
                device_id=(tgt,), device_id_type=pl.DeviceIdType.MESH,
            )
            r.start()
            rdmas.append(r)

        xv = x_ref[...]
        q = lax.dot_general(
            xv, wq_ref[...], (((2,), (0,)), ((), ())),
            preferred_element_type=jnp.float32,
        )

        for r in rdmas:
            r.wait()

        kl, km, kr = kl_ref[...], k_ref[...], kr_ref[...]
        vl, vm, vr = vl_ref[...], v_ref[...], vr_ref[...]

        qi = my * S_LOC + lax.broadcasted_iota(jnp.int32, (S_LOC, 3 * S_LOC), 0)
        col = lax.broadcasted_iota(jnp.int32, (S_LOC, 3 * S_LOC), 1)
        ki = jnp.where(
            col < S_LOC, left * S_LOC + col,
            jnp.where(col < 2 * S_LOC, my * S_LOC + col - S_LOC,
                      right * S_LOC + col - 2 * S_LOC))
        mask = jnp.abs(qi - ki) <= WINDOW

        ctx_heads = []
        for h in range(HQ):
            q_h = q[:, :, h * DH:(h + 1) * DH]
            k_h = jnp.concatenate(
                [kl[:, :, h, :], km[:, :, h, :], kr[:, :, h, :]], axis=1)
            v_h = jnp.concatenate(
                [vl[:, :, h, :], vm[:, :, h, :], vr[:, :, h, :]], axis=1)
            scores = lax.dot_general(
                q_h, k_h, (((2,), (2,)), ((0,), (0,))),
                preferred_element_type=jnp.float32,
            ) * SCALE
            scores = jnp.where(mask[None], scores, -1e9)
            m = jnp.max(scores, axis=-1, keepdims=True)
            w = jnp.exp(scores - m)
            w = w / jnp.sum(w, axis=-1, keepdims=True)
            ctx_heads.append(lax.dot_general(
                w, v_h, (((2,), (1,)), ((0,), (0,))),
                preferred_element_type=jnp.float32,
            ))

        ctx = jnp.concatenate(ctx_heads, axis=2)
        out_ref[...] = lax.dot_general(
            ctx, wo_ref[...], (((2,), (0,)), ((), ())),
            preferred_element_type=jnp.float32,
        )

    return pl.pallas_call(
        body,
        out_shape=jax.ShapeDtypeStruct((B, S_LOC, D_MODEL), jnp.float32),
        in_specs=[pl.BlockSpec(memory_space=pltpu.VMEM)] * 5,
        out_specs=pl.BlockSpec(memory_space=pltpu.VMEM),
        scratch_shapes=[
            pltpu.VMEM((B, S_LOC, HQ, DH), jnp.float32),
            pltpu.VMEM((B, S_LOC, HQ, DH), jnp.float32),
            pltpu.VMEM((B, S_LOC, HQ, DH), jnp.float32),
            pltpu.VMEM((B, S_LOC, HQ, DH), jnp.float32),
            pltpu.SemaphoreType.DMA((4,)),
            pltpu.SemaphoreType.DMA((4,)),
        ],
        compiler_params=pltpu.CompilerParams(collective_id=0),
    )(x, Wq, K_ext, V_ext, Wo)


# device time: 16867 ns/iter; 1.4487x vs baseline; 1.4487x over previous
import jax
import jax.numpy as jnp
from jax import lax
from jax.experimental import pallas as pl
from jax.experimental.pallas import tpu as pltpu

N_DEV = 8
B, S_LOC, D_MODEL = 2, 128, 512
HQ, DH = 4, 64
WINDOW = 128
SCALE = 0.125


def kernel(x, Wq, K_ext, V_ext, Wo):
    def body(x_ref, wq_ref, k_ref, v_ref, wo_ref, out_ref,
             ksend, vsend, kl_buf, kr_buf, vl_buf, vr_buf,
             send_sems, recv_sems):
        my = lax.axis_index("i")
        left = lax.rem(my + N_DEV - 1, N_DEV)
        right = lax.rem(my + 1, N_DEV)

        barrier_sem = pltpu.get_barrier_semaphore()
        for nbr in (left, right):
            pl.semaphore_signal(
                barrier_sem, inc=1,
                device_id=(nbr,), device_id_type=pl.DeviceIdType.MESH,
            )
        pl.semaphore_wait(barrier_sem, 2)

        ksend[...] = k_ref[...].astype(jnp.bfloat16)
        vsend[...] = v_ref[...].astype(jnp.bfloat16)
        rdmas = []
        for idx, (src, dst, tgt) in enumerate((
            (ksend, kl_buf, right),
            (ksend, kr_buf, left),
            (vsend, vl_buf, right),
            (vsend, vr_buf, left),
        )):
            r = pltpu.make_async_remote_copy(
                src_ref=src, dst_ref=dst,
                send_sem=send_sems.at[idx], recv_sem=recv_sems.at[idx],
                device_id=(tgt,), device_id_type=pl.DeviceIdType.MESH,
            )
            r.start()
            rdmas.append(r)
        rk_right, rk_left, rv_right, rv_left = rdmas

        q = lax.dot_general(
            x_ref[...].astype(jnp.bfloat16), wq_ref[...].astype(jnp.bfloat16),
            (((2,), (0,)), ((), ())),
            preferred_element_type=jnp.float32,
        ).astype(jnp.bfloat16)
        km = ksend[...]
        vm = vsend[...]
        sc_m = []
        for h in range(HQ):
            sc_m.append(lax.dot_general(
                q[:, :, h * DH:(h + 1) * DH], km[:, :, h, :],
                (((2,), (2,)), ((0,), (0,))),
                preferred_element_type=jnp.float32,
            ) * SCALE)

        qi = my * S_LOC + lax.broadcasted_iota(jnp.int32, (S_LOC, S_LOC), 0)
        jj = lax.broadcasted_iota(jnp.int32, (S_LOC, S_LOC), 1)
        mask_l = jnp.abs(qi - (left * S_LOC + jj)) <= WINDOW
        mask_r = jnp.abs(qi - (right * S_LOC + jj)) <= WINDOW

        rk_right.wait_recv()
        rk_left.wait_recv()
        kl = kl_buf[...]
        kr = kr_buf[...]
        probs = []
        for h in range(HQ):
            q_h = q[:, :, h * DH:(h + 1) * DH]
            sc_l = lax.dot_general(
                q_h, kl[:, :, h, :], (((2,), (2,)), ((0,), (0,))),
                preferred_element_type=jnp.float32) * SCALE
            sc_r = lax.dot_general(
                q_h, kr[:, :, h, :], (((2,), (2,)), ((0,), (0,))),
                preferred_element_type=jnp.float32) * SCALE
            sc_l = jnp.where(mask_l[None], sc_l, -1e9)
            sc_r = jnp.where(mask_r[None], sc_r, -1e9)
            sc = jnp.concatenate([sc_l, sc_m[h], sc_r], axis=2)
            m = jnp.max(sc, axis=-1, keepdims=True)
            w = jnp.exp(sc - m)
            w = w / jnp.sum(w, axis=-1, keepdims=True)
            probs.append(w.astype(jnp.bfloat16))

        rv_right.wait_recv()
        rv_left.wait_recv()
        vl = vl_buf[...]
        vr = vr_buf[...]
        ctxs = []
        for h in range(HQ):
            v_h = jnp.concatenate(
                [vl[:, :, h, :], vm[:, :, h, :], vr[:, :, h, :]], axis=1)
            ctxs.append(lax.dot_general(
                probs[h], v_h, (((2,), (1,)), ((0,), (0,))),
                preferred_element_type=jnp.float32))
        ctx = jnp.concatenate(ctxs, axis=2).astype(jnp.bfloat16)
        out_ref[...] = lax.dot_general(
            ctx, wo_ref[...].astype(jnp.bfloat16),
            (((2,), (0,)), ((), ())),
            preferred_element_type=jnp.float32,
        )

        for r in rdmas:
            r.wait_send()

    return pl.pallas_call(
        body,
        out_shape=jax.ShapeDtypeStruct((B, S_LOC, D_MODEL), jnp.float32),
        in_specs=[pl.BlockSpec(memory_space=pltpu.VMEM)] * 5,
        out_specs=pl.BlockSpec(memory_space=pltpu.VMEM),
        scratch_shapes=[
            pltpu.VMEM((B, S_LOC, HQ, DH), jnp.bfloat16),
            pltpu.VMEM((B, S_LOC, HQ, DH), jnp.bfloat16),
            pltpu.VMEM((B, S_LOC, HQ, DH), jnp.bfloat16),
            pltpu.VMEM((B, S_LOC, HQ, DH), jnp.bfloat16),
            pltpu.VMEM((B, S_LOC, HQ, DH), jnp.bfloat16),
            pltpu.VMEM((B, S_LOC, HQ, DH), jnp.bfloat16),
            pltpu.SemaphoreType.DMA((4,)),
            pltpu.SemaphoreType.DMA((4,)),
        ],
        compiler_params=pltpu.CompilerParams(collective_id=0),
    )(x, Wq, K_ext, V_ext, Wo)
